# baseline (speedup 1.0000x reference)
import functools

import jax
import jax.numpy as jnp
from jax.experimental import pallas as pl
from jax.experimental.pallas import tpu as pltpu

_BF16 = jnp.bfloat16


def _gn_silu_bf16(xf, gamma, beta, num_groups, eps, mask, count, mask_input):
    _, C = xf.shape
    cpg = C // num_groups
    denom = jnp.float32(count * cpg)

    xm = xf * mask if mask_input else xf
    csum = jnp.sum(xm, axis=0, keepdims=True)
    csq = jnp.sum(xf * xm, axis=0, keepdims=True)
    lane = jax.lax.broadcasted_iota(jnp.int32, (1, C), 1)

    def _seg_allsum(v):
        s = 1
        while s < cpg:
            partner = jnp.where((lane & s) == 0,
                                jnp.roll(v, -s, axis=1),
                                jnp.roll(v, s, axis=1))
            v = v + partner
            s *= 2
        return v

    mean_c = _seg_allsum(csum) / denom
    ex2_c = _seg_allsum(csq) / denom
    var_c = jnp.maximum(ex2_c - mean_c * mean_c, 0.0)
    inv_c = jax.lax.rsqrt(var_c + eps)
    scale = inv_c * gamma
    shift = beta - mean_c * scale
    y = xf * scale + shift
    y = y * jax.nn.sigmoid(y)
    if mask is not None:
        y = y * mask
    return y.astype(_BF16)


def _store_shifted(xp_ref, ym, C, shifts, KT, OFF, FR, SR):
    xp_ref[...] = jnp.zeros(xp_ref.shape, xp_ref.dtype)
    for j, d in enumerate(shifts):
        xp_ref[OFF - d:OFF - d + SR, j * C:(j + 1) * C] = ym
    if KT > 1:
        rep = xp_ref[(KT - 1) * FR:KT * FR, :]
        for f in range(KT - 1):
            xp_ref[f * FR:(f + 1) * FR, :] = rep


def _mrb_conv(pairs, M):
    per_mxu = [0, 0]
    for i, (lhs, rhs) in enumerate(pairs):
        mx = i % 2
        sr = per_mxu[mx] % 2
        pltpu.matmul_push_rhs(rhs, staging_register=sr, mxu_index=mx)
        pltpu.matmul_acc_lhs(acc_addr=0, lhs=lhs, mxu_index=mx,
                             load_staged_rhs=sr)
        per_mxu[mx] += 1
    r0 = pltpu.matmul_pop(acc_addr=0, shape=(M, 256), dtype=jnp.float32,
                          mxu_index=0)
    r1 = pltpu.matmul_pop(acc_addr=0, shape=(M, 256), dtype=jnp.float32,
                          mxu_index=1)
    return r0 + r1


def _conv_pairs(xp_ref, w_ref, bases, SR):
    n_lblk = xp_ref.shape[-1] // 256
    pairs = []
    t_idx = 0
    for base in bases:
        for b in range(n_lblk):
            pairs.append(
                (xp_ref[base:base + SR, b * 256:(b + 1) * 256],
                 w_ref[t_idx * 256:(t_idx + 1) * 256, :]))
            t_idx += 1
    return pairs


def _block_kernel(xg_ref, g1_ref, b1_ref, w1_ref, cb1_ref, g2_ref, b2_ref,
                  w2_ref, cb2_ref, ninw_ref, o_ref, *scratch,
                  num_groups, eps, T, H, W, WP, KS, Cin, Cmid, Cout,
                  PER_STEP):
    KT, KH, KW = KS
    HP = H + 2 * (KH // 2)
    FR = HP * WP
    SR = T * FR
    S = T * H * W
    OFF = (KT - 1) * FR + (KH // 2) * WP + (KW // 2)

    r = jax.lax.broadcasted_iota(jnp.int32, (SR, 1), 0)
    mask = ((r % WP < W) & (r % FR < H * WP)).astype(jnp.float32)

    sh1 = sh2 = list(range(KW))
    bases1 = bases2 = [kt * FR + kh * WP
                       for kt in range(KT) for kh in range(KH)]

    xs = [xg_ref[i] for i in range(PER_STEP)]
    hs = []
    for i in range(PER_STEP):
        y1 = _gn_silu_bf16(xs[i], g1_ref[...], b1_ref[...], num_groups,
                           eps, mask, S, mask_input=False)
        _store_shifted(scratch[i], y1, Cin, sh1, KT, OFF, FR, SR)
        h = _mrb_conv(_conv_pairs(scratch[i], w1_ref, bases1, SR), SR)
        hs.append(h + cb1_ref[...])

    for i in range(PER_STEP):
        y2 = _gn_silu_bf16(hs[i], g2_ref[...], b2_ref[...], num_groups,
                           eps, mask, S, mask_input=True)
        xp2 = scratch[PER_STEP + i]
        _store_shifted(xp2, y2, Cmid, sh2, KT, OFF, FR, SR)
        pairs = _conv_pairs(xp2, w2_ref, bases2, SR)
        xb = xs[i].astype(_BF16)
        if Cin < 256:
            xb = jnp.concatenate(
                [xb, jnp.zeros((SR, 256 - Cin), _BF16)], axis=-1)
        pairs.append((xb, ninw_ref[...]))
        acc = _mrb_conv(pairs, SR) + cb2_ref[...]

        o4 = acc.reshape(T, HP, WP, Cout)[:, :H, :W, :]
        o_ref[i] = o4.reshape(S, Cout).astype(o_ref.dtype).T


def kernel(x, norm1_gamma, norm1_beta, conv1_w, conv1_b, norm2_gamma,
           norm2_beta, conv2_w, conv2_b, nin_w, nin_b):
    N, Cin, T, H, W = x.shape
    S = T * H * W
    KT, KH, KW, _, Cmid = conv1_w.shape
    Cout = conv2_w.shape[-1]
    num_groups, eps = 32, 1e-6

    HP = H + 2 * (KH // 2)
    WP = ((W + 2 * (KW // 2) + 7) // 8) * 8
    FR = HP * WP
    SR = T * FR
    maxsh = (KH - 1) * WP + (KW - 1)
    RTOT = ((KT - 1) * FR + (KH - 1) * WP + SR + maxsh + 7) // 8 * 8

    lw1 = ((KW * Cin + 255) // 256) * 256
    lw2 = ((KW * Cmid + 255) // 256) * 256

    xt = jnp.transpose(x, (0, 2, 3, 4, 1))
    xg = jnp.pad(xt, ((0, 0), (0, 0), (0, HP - H), (0, WP - W), (0, 0)))
    xg = xg.reshape(N, SR, Cin)

    def _tile_weights(w, ngrp, lw):
        co = w.shape[-1]
        wg = w.astype(_BF16).reshape(ngrp, -1, co)
        wg = jnp.pad(wg, ((0, 0), (0, lw - wg.shape[1]), (0, 0)))
        return wg.reshape(-1, co)

    w1e = _tile_weights(conv1_w, KT * KH, lw1)
    w2e = _tile_weights(conv2_w, KT * KH, lw2)
    nine = jnp.concatenate(
        [nin_w.astype(_BF16),
         jnp.zeros((256 - Cin, Cout), _BF16)], axis=0) if Cin < 256 else \
        nin_w.astype(_BF16)
    cb2 = (conv2_b + nin_b).astype(jnp.float32).reshape(1, Cout)

    PER_STEP = 4 if N % 4 == 0 else (2 if N % 2 == 0 else 1)
    body = functools.partial(
        _block_kernel, num_groups=num_groups, eps=eps, T=T, H=H, W=W,
        WP=WP, KS=(KT, KH, KW), Cin=Cin, Cmid=Cmid, Cout=Cout,
        PER_STEP=PER_STEP)

    out = pl.pallas_call(
        body,
        out_shape=jax.ShapeDtypeStruct((N, Cout, S), x.dtype),
        grid=(N // PER_STEP,),
        in_specs=[
            pl.BlockSpec((PER_STEP, SR, Cin), lambda n: (n, 0, 0)),
            pl.BlockSpec((1, Cin), lambda n: (0, 0)),
            pl.BlockSpec((1, Cin), lambda n: (0, 0)),
            pl.BlockSpec(w1e.shape, lambda n: (0, 0)),
            pl.BlockSpec((1, Cmid), lambda n: (0, 0)),
            pl.BlockSpec((1, Cmid), lambda n: (0, 0)),
            pl.BlockSpec((1, Cmid), lambda n: (0, 0)),
            pl.BlockSpec(w2e.shape, lambda n: (0, 0)),
            pl.BlockSpec((1, Cout), lambda n: (0, 0)),
            pl.BlockSpec((256, Cout), lambda n: (0, 0)),
        ],
        out_specs=pl.BlockSpec((PER_STEP, Cout, S), lambda n: (n, 0, 0)),
        scratch_shapes=(
            [pltpu.VMEM((RTOT, lw1), _BF16) for _ in range(PER_STEP)]
            + [pltpu.VMEM((RTOT, lw2), _BF16) for _ in range(PER_STEP)]),
        compiler_params=pltpu.CompilerParams(
            dimension_semantics=("parallel",),
            vmem_limit_bytes=100 * 1024 * 1024,
        ),
    )(xg, norm1_gamma.reshape(1, Cin).astype(jnp.float32),
      norm1_beta.reshape(1, Cin).astype(jnp.float32), w1e,
      conv1_b.astype(jnp.float32).reshape(1, Cmid),
      norm2_gamma.reshape(1, Cmid).astype(jnp.float32),
      norm2_beta.reshape(1, Cmid).astype(jnp.float32), w2e, cb2, nine)

    return out.reshape(N, Cout, T, H, W)


# speedup vs baseline: 1.0283x; 1.0283x over previous
import functools

import jax
import jax.numpy as jnp
from jax.experimental import pallas as pl
from jax.experimental.pallas import tpu as pltpu

_BF16 = jnp.bfloat16


def _gn_silu_bf16(xf, gamma, beta, num_groups, eps, mask, count, mask_input):
    _, C = xf.shape
    cpg = C // num_groups
    denom = jnp.float32(count * cpg)

    xm = xf * mask if mask_input else xf
    csum = jnp.sum(xm, axis=0, keepdims=True)
    csq = jnp.sum(xf * xm, axis=0, keepdims=True)
    lane = jax.lax.broadcasted_iota(jnp.int32, (1, C), 1)

    def _seg_allsum(v):
        s = 1
        while s < cpg:
            partner = jnp.where((lane & s) == 0,
                                jnp.roll(v, -s, axis=1),
                                jnp.roll(v, s, axis=1))
            v = v + partner
            s *= 2
        return v

    mean_c = _seg_allsum(csum) / denom
    ex2_c = _seg_allsum(csq) / denom
    var_c = jnp.maximum(ex2_c - mean_c * mean_c, 0.0)
    inv_c = jax.lax.rsqrt(var_c + eps)
    scale = inv_c * gamma
    shift = beta - mean_c * scale
    y = xf * scale + shift
    y = y * jax.nn.sigmoid(y)
    if mask is not None:
        y = y * mask
    return y.astype(_BF16)


def _store_shifted(xp_ref, ym, C, shifts, KT, OFF, FR, SR):
    xp_ref[...] = jnp.zeros(xp_ref.shape, xp_ref.dtype)
    for j, d in enumerate(shifts):
        xp_ref[OFF - d:OFF - d + SR, j * C:(j + 1) * C] = ym
    if KT > 1:
        rep = xp_ref[(KT - 1) * FR:KT * FR, :]
        for f in range(KT - 1):
            xp_ref[f * FR:(f + 1) * FR, :] = rep


def _mrb_conv(pairs, M):
    per_mxu = [0, 0]
    for i, (lhs, rhs) in enumerate(pairs):
        mx = i % 2
        sr = per_mxu[mx] % 2
        pltpu.matmul_push_rhs(rhs, staging_register=sr, mxu_index=mx)
        pltpu.matmul_acc_lhs(acc_addr=0, lhs=lhs, mxu_index=mx,
                             load_staged_rhs=sr)
        per_mxu[mx] += 1
    r0 = pltpu.matmul_pop(acc_addr=0, shape=(M, 256), dtype=jnp.float32,
                          mxu_index=0)
    r1 = pltpu.matmul_pop(acc_addr=0, shape=(M, 256), dtype=jnp.float32,
                          mxu_index=1)
    return r0 + r1


def _conv_pairs(xp_ref, w_ref, bases, SR):
    n_lblk = xp_ref.shape[-1] // 256
    pairs = []
    t_idx = 0
    for base in bases:
        for b in range(n_lblk):
            pairs.append(
                (xp_ref[base:base + SR, b * 256:(b + 1) * 256],
                 w_ref[t_idx * 256:(t_idx + 1) * 256, :]))
            t_idx += 1
    return pairs


def _block_kernel(xg_ref, g1_ref, b1_ref, w1_ref, cb1_ref, g2_ref, b2_ref,
                  w2_ref, cb2_ref, ninw_ref, o_ref, *scratch,
                  num_groups, eps, T, H, W, WP, KS, Cin, Cmid, Cout,
                  PER_STEP):
    KT, KH, KW = KS
    HP = H + 2 * (KH // 2)
    FR = HP * WP
    SR = T * FR
    S = T * H * W
    OFF = (KT - 1) * FR + (KH // 2) * WP + (KW // 2)

    r = jax.lax.broadcasted_iota(jnp.int32, (SR, 1), 0)
    mask = ((r % WP < W) & (r % FR < H * WP)).astype(jnp.float32)

    sh1 = sh2 = list(range(KW))
    bases1 = bases2 = [kt * FR + kh * WP
                       for kt in range(KT) for kh in range(KH)]

    xs = [xg_ref[i] for i in range(PER_STEP)]
    hs = []
    for i in range(PER_STEP):
        y1 = _gn_silu_bf16(xs[i], g1_ref[...], b1_ref[...], num_groups,
                           eps, mask, S, mask_input=False)
        _store_shifted(scratch[i], y1, Cin, sh1, KT, OFF, FR, SR)
        h = _mrb_conv(_conv_pairs(scratch[i], w1_ref, bases1, SR), SR)
        hs.append(h + cb1_ref[...])

    for i in range(PER_STEP):
        y2 = _gn_silu_bf16(hs[i], g2_ref[...], b2_ref[...], num_groups,
                           eps, mask, S, mask_input=True)
        xp2 = scratch[PER_STEP + i]
        _store_shifted(xp2, y2, Cmid, sh2, KT, OFF, FR, SR)
        pairs = _conv_pairs(xp2, w2_ref, bases2, SR)
        xb = xs[i].astype(_BF16)
        if Cin < 256:
            xb = jnp.concatenate(
                [xb, jnp.zeros((SR, 256 - Cin), _BF16)], axis=-1)
        pairs.append((xb, ninw_ref[...]))
        acc = _mrb_conv(pairs, SR) + cb2_ref[...]

        o4 = acc.reshape(T, HP, WP, Cout)[:, :H, :W, :]
        o_ref[i] = o4.reshape(S, Cout).astype(o_ref.dtype).T


def kernel(x, norm1_gamma, norm1_beta, conv1_w, conv1_b, norm2_gamma,
           norm2_beta, conv2_w, conv2_b, nin_w, nin_b):
    N, Cin, T, H, W = x.shape
    S = T * H * W
    KT, KH, KW, _, Cmid = conv1_w.shape
    Cout = conv2_w.shape[-1]
    num_groups, eps = 32, 1e-6

    HP = H + 2 * (KH // 2)
    WP = ((W + 2 * (KW // 2) + 7) // 8) * 8
    FR = HP * WP
    SR = T * FR
    maxsh = (KH - 1) * WP + (KW - 1)
    RTOT = ((KT - 1) * FR + (KH - 1) * WP + SR + maxsh + 7) // 8 * 8

    lw1 = ((KW * Cin + 255) // 256) * 256
    lw2 = ((KW * Cmid + 255) // 256) * 256

    xt = jnp.transpose(x, (0, 2, 3, 4, 1))
    xg = jnp.pad(xt, ((0, 0), (0, 0), (0, HP - H), (0, WP - W), (0, 0)))
    xg = xg.reshape(N, SR, Cin)

    def _tile_weights(w, ngrp, lw):
        co = w.shape[-1]
        wg = w.astype(_BF16).reshape(ngrp, -1, co)
        wg = jnp.pad(wg, ((0, 0), (0, lw - wg.shape[1]), (0, 0)))
        return wg.reshape(-1, co)

    w1e = _tile_weights(conv1_w, KT * KH, lw1)
    w2e = _tile_weights(conv2_w, KT * KH, lw2)
    nine = jnp.concatenate(
        [nin_w.astype(_BF16),
         jnp.zeros((256 - Cin, Cout), _BF16)], axis=0) if Cin < 256 else \
        nin_w.astype(_BF16)
    cb2 = (conv2_b + nin_b).astype(jnp.float32).reshape(1, Cout)

    PER_STEP = 2 if N % 2 == 0 else 1
    body = functools.partial(
        _block_kernel, num_groups=num_groups, eps=eps, T=T, H=H, W=W,
        WP=WP, KS=(KT, KH, KW), Cin=Cin, Cmid=Cmid, Cout=Cout,
        PER_STEP=PER_STEP)

    out = pl.pallas_call(
        body,
        out_shape=jax.ShapeDtypeStruct((N, Cout, S), x.dtype),
        grid=(N // PER_STEP,),
        in_specs=[
            pl.BlockSpec((PER_STEP, SR, Cin), lambda n: (n, 0, 0)),
            pl.BlockSpec((1, Cin), lambda n: (0, 0)),
            pl.BlockSpec((1, Cin), lambda n: (0, 0)),
            pl.BlockSpec(w1e.shape, lambda n: (0, 0)),
            pl.BlockSpec((1, Cmid), lambda n: (0, 0)),
            pl.BlockSpec((1, Cmid), lambda n: (0, 0)),
            pl.BlockSpec((1, Cmid), lambda n: (0, 0)),
            pl.BlockSpec(w2e.shape, lambda n: (0, 0)),
            pl.BlockSpec((1, Cout), lambda n: (0, 0)),
            pl.BlockSpec((256, Cout), lambda n: (0, 0)),
        ],
        out_specs=pl.BlockSpec((PER_STEP, Cout, S), lambda n: (n, 0, 0)),
        scratch_shapes=(
            [pltpu.VMEM((RTOT, lw1), _BF16) for _ in range(PER_STEP)]
            + [pltpu.VMEM((RTOT, lw2), _BF16) for _ in range(PER_STEP)]),
        compiler_params=pltpu.CompilerParams(
            dimension_semantics=("parallel",),
            vmem_limit_bytes=100 * 1024 * 1024,
        ),
    )(xg, norm1_gamma.reshape(1, Cin).astype(jnp.float32),
      norm1_beta.reshape(1, Cin).astype(jnp.float32), w1e,
      conv1_b.astype(jnp.float32).reshape(1, Cmid),
      norm2_gamma.reshape(1, Cmid).astype(jnp.float32),
      norm2_beta.reshape(1, Cmid).astype(jnp.float32), w2e, cb2, nine)

    return out.reshape(N, Cout, T, H, W)
